# baseline (speedup 1.0000x reference)
import functools

import jax
import jax.numpy as jnp
from jax.experimental import pallas as pl

E = 320000
N = 10000
D = 128

_BR = 1000


def _proj_body(x_ref, wi_ref, wu_ref, wd_ref, xi_ref, xu_ref, xd_ref):
    x = x_ref[...]
    xi_ref[...] = jnp.dot(x, wi_ref[...], preferred_element_type=jnp.float32)
    xu_ref[...] = jnp.dot(x, wu_ref[...], preferred_element_type=jnp.float32)
    xd_ref[...] = jnp.dot(x, wd_ref[...], preferred_element_type=jnp.float32)


def _project(x_1, W_id, W_up, W_down):
    grid = (E // _BR,)
    bs_x = pl.BlockSpec((_BR, D), lambda i: (i, 0))
    bs_w = pl.BlockSpec((D, D), lambda i: (0, 0))
    return pl.pallas_call(
        _proj_body,
        grid=grid,
        in_specs=[bs_x, bs_w, bs_w, bs_w],
        out_specs=[bs_x, bs_x, bs_x],
        out_shape=[jax.ShapeDtypeStruct((E, D), jnp.float32)] * 3,
    )(x_1, W_id, W_up, W_down)


def _relu_sum_body(a_ref, b_ref, c_ref, o_ref):
    o_ref[...] = jnp.maximum(a_ref[...] + b_ref[...] + c_ref[...], 0.0)


def _relu_sum(a, b, c):
    grid = (E // _BR,)
    bs = pl.BlockSpec((_BR, D), lambda i: (i, 0))
    return pl.pallas_call(
        _relu_sum_body,
        grid=grid,
        in_specs=[bs, bs, bs],
        out_specs=bs,
        out_shape=jax.ShapeDtypeStruct((E, D), jnp.float32),
    )(a, b, c)


def _spmm(rows, cols, vals, x, n_rows):
    return jax.ops.segment_sum(vals[:, None] * x[cols], rows, num_segments=n_rows)


def kernel(x_1, lap_up_indices, lap_up_values, lap_down_indices, lap_down_values,
           inc_rows, inc_cols, inc_values, y, W_id, W_up, W_down):
    xi, xu, xd = _project(x_1, W_id, W_up, W_down)
    h_up = _spmm(lap_up_indices[0], lap_up_indices[1], lap_up_values, xu, E)
    h_down = _spmm(lap_down_indices[0], lap_down_indices[1], lap_down_values, xd, E)
    h = _relu_sum(xi, h_up, h_down)
    x_0 = _spmm(inc_rows, inc_cols, inc_values, h, N)
    return (x_0, y)


# speedup vs baseline: 1.1180x; 1.1180x over previous
import functools

import jax
import jax.numpy as jnp
from jax import lax
from jax.experimental import pallas as pl
from jax.experimental.pallas import tpu as pltpu
from jax.experimental.pallas import tpu_sc as plsc

E = 320000
N = 10000
D = 128

NC = 2
NS = 16
NW = NC * NS

_W = 128
_TI = 160
_PAD_I = NW * _TI * _W
_NP = 10240


def _inc_body(h_hbm, rows_hbm, cols_hbm, vals_hbm, out_hbm,
              acc, ibr, ibc, ibv, gbuf, zbuf,
              slin0, slin1, slin2, slin3, sg0, sg1, ss0, ss1):
    cid = lax.axis_index("c")
    sid = lax.axis_index("s")
    wid = sid * NC + cid
    base = wid * _TI * _W

    slin = (slin0, slin1, slin2, slin3)
    sg = (sg0, sg1)
    ss = (ss0, ss1)

    @pl.loop(0, 64)
    def _(i):
        for k in range(8):
            zbuf[i, pl.ds(16 * k, 16)] = jnp.zeros((16,), jnp.float32)
    for c in range(10):
        pltpu.sync_copy(zbuf, acc.at[pl.ds(sid * 640 + c * 64, 64)])
    plsc.subcore_barrier()

    def lin_start(w, s):
        off = base + w * _W
        pltpu.async_copy(rows_hbm.at[pl.ds(off, _W)], ibr.at[s], slin[s])
        pltpu.async_copy(cols_hbm.at[pl.ds(off, _W)], ibc.at[s], slin[s])
        pltpu.async_copy(vals_hbm.at[pl.ds(off, _W)], ibv.at[s], slin[s])

    def lin_wait(w, s):
        off = base + w * _W
        pltpu.make_async_copy(rows_hbm.at[pl.ds(off, _W)], ibr.at[s], slin[s]).wait()
        pltpu.make_async_copy(cols_hbm.at[pl.ds(off, _W)], ibc.at[s], slin[s]).wait()
        pltpu.make_async_copy(vals_hbm.at[pl.ds(off, _W)], ibv.at[s], slin[s]).wait()

    def gat_start(b, g):
        pltpu.async_copy(h_hbm.at[ibc.at[b]], gbuf.at[g], sg[g])

    def gat_wait(b, g):
        pltpu.make_async_copy(h_hbm.at[ibc.at[b]], gbuf.at[g], sg[g]).wait()

    def scat_start(b, g):
        pltpu.async_copy(gbuf.at[g], acc.at[ibr.at[b]], ss[g], add=True)

    def scat_wait(b, g):
        pltpu.make_async_copy(gbuf.at[g], acc.at[ibr.at[b]], ss[g]).wait()

    def mul(b, g):
        @pl.loop(0, _W)
        def _(j):
            vs = plsc.load_gather(ibv.at[b], [jnp.full((16,), j, jnp.int32)])
            for k in range(8):
                gbuf.at[g][j, pl.ds(16 * k, 16)] = (
                    gbuf.at[g][j, pl.ds(16 * k, 16)] * vs)

    lin_start(0, 0)
    lin_start(1, 1)
    lin_wait(0, 0)
    gat_start(0, 0)

    @pl.loop(0, _TI, step=4)
    def _(w0):
        for s in range(4):
            w = w0 + s
            b, g = s, s % 2
            gat_wait(b, g)

            @pl.when(w >= 1)
            def _():
                scat_wait((s + 3) % 4, (g + 1) % 2)

            @pl.when(w + 1 < _TI)
            def _():
                lin_wait(w + 1, (s + 1) % 4)
                gat_start((s + 1) % 4, (g + 1) % 2)

            mul(b, g)
            scat_start(b, g)

            @pl.when(w + 2 < _TI)
            def _():
                lin_start(w + 2, (s + 2) % 4)

    scat_wait((_TI - 1) % 4, (_TI - 1) % 2)
    plsc.subcore_barrier()

    for c in range(5):
        off = sid * 640 + c * 128
        pltpu.sync_copy(acc.at[pl.ds(off, 128)],
                        out_hbm.at[pl.ds(cid * _NP + off, 128)])


_inc_call = pl.kernel(
    _inc_body,
    out_type=jax.ShapeDtypeStruct((NC * _NP, D), jnp.float32),
    mesh=plsc.VectorSubcoreMesh(core_axis_name="c", subcore_axis_name="s",
                                num_cores=NC, num_subcores=NS),
    compiler_params=pltpu.CompilerParams(needs_layout_passes=False),
    scratch_types=[
        pltpu.VMEM_SHARED((_NP, D), jnp.float32),
        pltpu.VMEM((4, _W), jnp.int32),
        pltpu.VMEM((4, _W), jnp.int32),
        pltpu.VMEM((4, _W), jnp.float32),
        pltpu.VMEM((2, _W, D), jnp.float32),
        pltpu.VMEM((64, D), jnp.float32),
        pltpu.SemaphoreType.DMA, pltpu.SemaphoreType.DMA, pltpu.SemaphoreType.DMA,
        pltpu.SemaphoreType.DMA, pltpu.SemaphoreType.DMA, pltpu.SemaphoreType.DMA,
        pltpu.SemaphoreType.DMA, pltpu.SemaphoreType.DMA,
    ],
)


def _pad_coo(rows, cols, vals, n_pad, row_mod):
    pad = n_pad - rows.shape[0]
    ar = jnp.arange(pad, dtype=jnp.int32)
    rows = jnp.concatenate([rows.astype(jnp.int32), ar % row_mod])
    cols = jnp.concatenate([cols.astype(jnp.int32), ar % E])
    vals = jnp.concatenate([vals, jnp.zeros((pad,), jnp.float32)])
    return rows, cols, vals


_BR = 1000


def _proj_body(x_ref, wi_ref, wu_ref, wd_ref, xi_ref, xu_ref, xd_ref):
    x = x_ref[...]
    xi_ref[...] = jnp.dot(x, wi_ref[...], preferred_element_type=jnp.float32)
    xu_ref[...] = jnp.dot(x, wu_ref[...], preferred_element_type=jnp.float32)
    xd_ref[...] = jnp.dot(x, wd_ref[...], preferred_element_type=jnp.float32)


def _project(x_1, W_id, W_up, W_down):
    grid = (E // _BR,)
    bs_x = pl.BlockSpec((_BR, D), lambda i: (i, 0))
    bs_w = pl.BlockSpec((D, D), lambda i: (0, 0))
    return pl.pallas_call(
        _proj_body,
        grid=grid,
        in_specs=[bs_x, bs_w, bs_w, bs_w],
        out_specs=[bs_x, bs_x, bs_x],
        out_shape=[jax.ShapeDtypeStruct((E, D), jnp.float32)] * 3,
    )(x_1, W_id, W_up, W_down)


def _relu_sum_body(a_ref, b_ref, c_ref, o_ref):
    o_ref[...] = jnp.maximum(a_ref[...] + b_ref[...] + c_ref[...], 0.0)


def _relu_sum(a, b, c):
    grid = (E // _BR,)
    bs = pl.BlockSpec((_BR, D), lambda i: (i, 0))
    return pl.pallas_call(
        _relu_sum_body,
        grid=grid,
        in_specs=[bs, bs, bs],
        out_specs=bs,
        out_shape=jax.ShapeDtypeStruct((E, D), jnp.float32),
    )(a, b, c)


def _add2_body(a_ref, b_ref, o_ref):
    o_ref[...] = a_ref[...] + b_ref[...]


def _add_partials(p):
    grid = (N // 1000,)
    bs = pl.BlockSpec((1000, D), lambda i: (i, 0))
    return pl.pallas_call(
        _add2_body,
        grid=grid,
        in_specs=[bs, bs],
        out_specs=bs,
        out_shape=jax.ShapeDtypeStruct((N, D), jnp.float32),
    )(p[:N], p[_NP:_NP + N])


def _spmm(rows, cols, vals, x, n_rows):
    return jax.ops.segment_sum(vals[:, None] * x[cols], rows, num_segments=n_rows)


def kernel(x_1, lap_up_indices, lap_up_values, lap_down_indices, lap_down_values,
           inc_rows, inc_cols, inc_values, y, W_id, W_up, W_down):
    xi, xu, xd = _project(x_1, W_id, W_up, W_down)
    h_up = _spmm(lap_up_indices[0], lap_up_indices[1], lap_up_values, xu, E)
    h_down = _spmm(lap_down_indices[0], lap_down_indices[1], lap_down_values, xd, E)
    h = _relu_sum(xi, h_up, h_down)

    ir, ic, iv = _pad_coo(inc_rows, inc_cols, inc_values, _PAD_I, N)
    partials = _inc_call(h, ir, ic, iv)
    x_0 = _add_partials(partials)
    return (x_0, y)


# speedup vs baseline: 1.4340x; 1.2827x over previous
import functools

import jax
import jax.numpy as jnp
from jax import lax
from jax.experimental import pallas as pl
from jax.experimental.pallas import tpu as pltpu
from jax.experimental.pallas import tpu_sc as plsc

E = 320000
N = 10000
D = 128

NC = 2
NS = 16
NW = NC * NS

_W = 128
_TI = 160
_PAD_I = NW * _TI * _W
_NP = 10240


def _inc_body(h_hbm, rows_hbm, cols_hbm, vals_hbm, out_hbm,
              acc, ibr, ibc, ibv, gbuf, zbuf,
              slin0, slin1, slin2, slin3, sg0, sg1, ss0, ss1):
    cid = lax.axis_index("c")
    sid = lax.axis_index("s")
    wid = sid * NC + cid
    base = wid * _TI * _W

    slin = (slin0, slin1, slin2, slin3)
    sg = (sg0, sg1)
    ss = (ss0, ss1)

    @pl.loop(0, 64)
    def _(i):
        for k in range(8):
            zbuf[i, pl.ds(16 * k, 16)] = jnp.zeros((16,), jnp.float32)
    for c in range(10):
        pltpu.sync_copy(zbuf, acc.at[pl.ds(sid * 640 + c * 64, 64)])
    plsc.subcore_barrier()

    def lin_start(w, s):
        off = base + w * _W
        pltpu.async_copy(rows_hbm.at[pl.ds(off, _W)], ibr.at[s], slin[s])
        pltpu.async_copy(cols_hbm.at[pl.ds(off, _W)], ibc.at[s], slin[s])
        pltpu.async_copy(vals_hbm.at[pl.ds(off, _W)], ibv.at[s], slin[s])

    def lin_wait(w, s):
        off = base + w * _W
        pltpu.make_async_copy(rows_hbm.at[pl.ds(off, _W)], ibr.at[s], slin[s]).wait()
        pltpu.make_async_copy(cols_hbm.at[pl.ds(off, _W)], ibc.at[s], slin[s]).wait()
        pltpu.make_async_copy(vals_hbm.at[pl.ds(off, _W)], ibv.at[s], slin[s]).wait()

    def gat_start(b, g):
        pltpu.async_copy(h_hbm.at[ibc.at[b]], gbuf.at[g], sg[g])

    def gat_wait(b, g):
        pltpu.make_async_copy(h_hbm.at[ibc.at[b]], gbuf.at[g], sg[g]).wait()

    def scat_start(b, g):
        pltpu.async_copy(gbuf.at[g], acc.at[ibr.at[b]], ss[g], add=True)

    def scat_wait(b, g):
        pltpu.make_async_copy(gbuf.at[g], acc.at[ibr.at[b]], ss[g]).wait()

    def mul(b, g):
        @pl.loop(0, _W)
        def _(j):
            vs = plsc.load_gather(ibv.at[b], [jnp.full((16,), j, jnp.int32)])
            for k in range(8):
                gbuf.at[g][j, pl.ds(16 * k, 16)] = (
                    gbuf.at[g][j, pl.ds(16 * k, 16)] * vs)

    lin_start(0, 0)
    lin_start(1, 1)
    lin_wait(0, 0)
    gat_start(0, 0)

    @pl.loop(0, _TI, step=4)
    def _(w0):
        for s in range(4):
            w = w0 + s
            b, g = s, s % 2
            gat_wait(b, g)

            @pl.when(w >= 1)
            def _():
                scat_wait((s + 3) % 4, (g + 1) % 2)

            @pl.when(w + 1 < _TI)
            def _():
                lin_wait(w + 1, (s + 1) % 4)
                gat_start((s + 1) % 4, (g + 1) % 2)

            mul(b, g)
            scat_start(b, g)

            @pl.when(w + 2 < _TI)
            def _():
                lin_start(w + 2, (s + 2) % 4)

    scat_wait((_TI - 1) % 4, (_TI - 1) % 2)
    plsc.subcore_barrier()

    for c in range(5):
        off = sid * 640 + c * 128
        pltpu.sync_copy(acc.at[pl.ds(off, 128)],
                        out_hbm.at[pl.ds(cid * _NP + off, 128)])


_inc_call = pl.kernel(
    _inc_body,
    out_type=jax.ShapeDtypeStruct((NC * _NP, D), jnp.float32),
    mesh=plsc.VectorSubcoreMesh(core_axis_name="c", subcore_axis_name="s",
                                num_cores=NC, num_subcores=NS),
    compiler_params=pltpu.CompilerParams(needs_layout_passes=False),
    scratch_types=[
        pltpu.VMEM_SHARED((_NP, D), jnp.float32),
        pltpu.VMEM((4, _W), jnp.int32),
        pltpu.VMEM((4, _W), jnp.int32),
        pltpu.VMEM((4, _W), jnp.float32),
        pltpu.VMEM((2, _W, D), jnp.float32),
        pltpu.VMEM((64, D), jnp.float32),
        pltpu.SemaphoreType.DMA, pltpu.SemaphoreType.DMA, pltpu.SemaphoreType.DMA,
        pltpu.SemaphoreType.DMA, pltpu.SemaphoreType.DMA, pltpu.SemaphoreType.DMA,
        pltpu.SemaphoreType.DMA, pltpu.SemaphoreType.DMA,
    ],
)


_WH = 1024
_NBK = 79
_NBP = 80
_CB = 4096
_TH = 98
_TP = 784
_PAD_L = 16 * _TH * _WH


def _bin_one(sid, rows, cols, vals, brow, bcol, bval, off_out,
             ghist, lhist, hist, offs, hbuf, pbr, pbc, pbv, dbuf, sbuf,
             sh, sl, ss):
    base = sid * (_PAD_L // 16)
    iota = lax.iota(jnp.int32, 16)
    one = jnp.ones((16,), jnp.int32)
    zero16 = jnp.zeros((16,), jnp.int32)

    @pl.loop(0, _NBP)
    def _(b):
        hist[pl.ds(b * 16, 16)] = jnp.zeros((16,), jnp.int32)

    def h_start(w, s):
        pltpu.async_copy(rows.at[pl.ds(base + w * _WH, _WH)],
                         hbuf.at[pl.ds(s * _WH, _WH)], sh[s])

    def h_wait(w, s):
        pltpu.make_async_copy(rows.at[pl.ds(base + w * _WH, _WH)],
                              hbuf.at[pl.ds(s * _WH, _WH)], sh[s]).wait()

    h_start(0, 0)
    h_start(1, 1)

    @pl.loop(0, _TH, step=2)
    def _(w0):
        for s in range(2):
            w = w0 + s
            h_wait(w, s)

            @pl.loop(0, _WH // 16)
            def _(i):
                r = hbuf[pl.ds(s * _WH + i * 16, 16)]
                bkt = r >> 12
                plsc.addupdate_scatter(hist, [bkt * 16 + iota], one)

            @pl.when(w + 2 < _TH)
            def _():
                h_start(w + 2, s)

    pltpu.sync_copy(hist, ghist.at[pl.ds(sid * 1280, 1280)])
    plsc.subcore_barrier()
    pltpu.sync_copy(ghist, lhist)

    @pl.loop(0, _NBP, init_carry=jnp.zeros((16,), jnp.int32))
    def _(b, tv):
        @pl.when(sid == 0)
        def _():
            plsc.store_scatter(sbuf, [jnp.full((16,), b, jnp.int32)], tv,
                               mask=iota == 0)
        for t in range(16):
            v = lhist[pl.ds(t * 1280 + b * 16, 16)]
            cs = plsc.cumsum(v)

            @pl.when(sid == t)
            def _():
                offs[pl.ds(b * 16, 16)] = tv + (cs - v)

            tv = tv + lax.reduce_sum(jnp.where(iota == 15, cs, 0), (0,))
        return tv

    @pl.when(sid == 0)
    def _():
        pltpu.sync_copy(sbuf, off_out)
    plsc.subcore_barrier()

    def p_start(w, s):
        off = base + w * 128
        pltpu.async_copy(rows.at[pl.ds(off, 128)], pbr.at[s], sl[s])
        pltpu.async_copy(cols.at[pl.ds(off, 128)], pbc.at[s], sl[s])
        pltpu.async_copy(vals.at[pl.ds(off, 128)], pbv.at[s], sl[s])

    def p_wait(w, s):
        off = base + w * 128
        pltpu.make_async_copy(rows.at[pl.ds(off, 128)], pbr.at[s], sl[s]).wait()
        pltpu.make_async_copy(cols.at[pl.ds(off, 128)], pbc.at[s], sl[s]).wait()
        pltpu.make_async_copy(vals.at[pl.ds(off, 128)], pbv.at[s], sl[s]).wait()

    def sc_start(s):
        pltpu.async_copy(pbr.at[s], brow.at[dbuf.at[s]], ss[s])
        pltpu.async_copy(pbc.at[s], bcol.at[dbuf.at[s]], ss[s])
        pltpu.async_copy(pbv.at[s], bval.at[dbuf.at[s]], ss[s])

    def sc_wait(s):
        pltpu.make_async_copy(pbr.at[s], brow.at[dbuf.at[s]], ss[s]).wait()
        pltpu.make_async_copy(pbc.at[s], bcol.at[dbuf.at[s]], ss[s]).wait()
        pltpu.make_async_copy(pbv.at[s], bval.at[dbuf.at[s]], ss[s]).wait()

    p_start(0, 0)
    p_start(1, 1)

    @pl.loop(0, _TP, step=4)
    def _(w0):
        for s in range(4):
            w = w0 + s
            p_wait(w, s)
            for k in range(8):
                r = pbr.at[s][pl.ds(16 * k, 16)]
                bkt = r >> 12
                dest = plsc.load_gather(offs, [bkt * 16 + iota])
                plsc.store_scatter(offs, [bkt * 16 + iota], dest + one)
                dbuf.at[s][pl.ds(16 * k, 16)] = dest
            sc_start(s)

            @pl.when(w >= 2)
            def _():
                sc_wait((s + 2) % 4)

            @pl.when(w + 2 < _TP)
            def _():
                p_start(w + 2, (s + 2) % 4)

    sc_wait((_TP - 2) % 4)
    sc_wait((_TP - 1) % 4)


def _bin_body(rows_u, cols_u, vals_u, rows_d, cols_d, vals_d,
              brow_u, bcol_u, bval_u, off_u, brow_d, bcol_d, bval_d, off_d,
              ghist, lhist, hist, offs, hbuf, pbr, pbc, pbv, dbuf, sbuf,
              sh0, sh1, sl0, sl1, sl2, sl3, ss0, ss1, ss2, ss3):
    cid = lax.axis_index("c")
    sid = lax.axis_index("s")
    sh = (sh0, sh1)
    sl = (sl0, sl1, sl2, sl3)
    ss = (ss0, ss1, ss2, ss3)

    @pl.when(cid == 0)
    def _():
        _bin_one(sid, rows_u, cols_u, vals_u, brow_u, bcol_u, bval_u, off_u,
                 ghist, lhist, hist, offs, hbuf, pbr, pbc, pbv, dbuf, sbuf,
                 sh, sl, ss)

    @pl.when(cid == 1)
    def _():
        _bin_one(sid, rows_d, cols_d, vals_d, brow_d, bcol_d, bval_d, off_d,
                 ghist, lhist, hist, offs, hbuf, pbr, pbc, pbv, dbuf, sbuf,
                 sh, sl, ss)


_bin_call = pl.kernel(
    _bin_body,
    out_type=[
        jax.ShapeDtypeStruct((_PAD_L,), jnp.int32),
        jax.ShapeDtypeStruct((_PAD_L,), jnp.int32),
        jax.ShapeDtypeStruct((_PAD_L,), jnp.float32),
        jax.ShapeDtypeStruct((128,), jnp.int32),
        jax.ShapeDtypeStruct((_PAD_L,), jnp.int32),
        jax.ShapeDtypeStruct((_PAD_L,), jnp.int32),
        jax.ShapeDtypeStruct((_PAD_L,), jnp.float32),
        jax.ShapeDtypeStruct((128,), jnp.int32),
    ],
    mesh=plsc.VectorSubcoreMesh(core_axis_name="c", subcore_axis_name="s",
                                num_cores=NC, num_subcores=NS),
    compiler_params=pltpu.CompilerParams(needs_layout_passes=False),
    scratch_types=[
        pltpu.VMEM_SHARED((NS * _NBP * 16,), jnp.int32),
        pltpu.VMEM((NS * _NBP * 16,), jnp.int32),
        pltpu.VMEM((_NBP * 16,), jnp.int32),
        pltpu.VMEM((_NBP * 16,), jnp.int32),
        pltpu.VMEM((2 * _WH,), jnp.int32),
        pltpu.VMEM((4, 128), jnp.int32),
        pltpu.VMEM((4, 128), jnp.int32),
        pltpu.VMEM((4, 128), jnp.float32),
        pltpu.VMEM((4, 128), jnp.int32),
        pltpu.VMEM((128,), jnp.int32),
        pltpu.SemaphoreType.DMA, pltpu.SemaphoreType.DMA,
        pltpu.SemaphoreType.DMA, pltpu.SemaphoreType.DMA,
        pltpu.SemaphoreType.DMA, pltpu.SemaphoreType.DMA,
        pltpu.SemaphoreType.DMA, pltpu.SemaphoreType.DMA,
        pltpu.SemaphoreType.DMA, pltpu.SemaphoreType.DMA,
    ],
)


def _ext(buf, j):
    iota = lax.iota(jnp.int32, 16)
    blk = (j >> 4) << 4
    v = buf[pl.ds(blk, 16)]
    return lax.reduce_sum(jnp.where(iota == (j & 15), v, 0), (0,))


def _seg_body(brow, bcol, bval, offv, xmat, out,
              acc, offbuf, cbuf, rbuf, vbuf, gbuf, zbuf,
              sl0, sl1, sl2, sl3, sg0, sg1, ss0, ss1, wo0, wo1):
    cid = lax.axis_index("c")
    sid = lax.axis_index("s")
    iota = lax.iota(jnp.int32, 16)
    sl = (sl0, sl1, sl2, sl3)
    sg = (sg0, sg1)
    ss = (ss0, ss1)
    wo = (wo0, wo1)

    pltpu.sync_copy(offv, offbuf)

    @pl.loop(0, 64)
    def _(i):
        for k in range(8):
            zbuf[i, pl.ds(16 * k, 16)] = jnp.zeros((16,), jnp.float32)

    def lin_start(w, g):
        off = w * 128
        pltpu.async_copy(bcol.at[pl.ds(off, 128)], cbuf.at[g], sl[g])
        pltpu.async_copy(brow.at[pl.ds(off, 128)], rbuf.at[g], sl[g])
        pltpu.async_copy(bval.at[pl.ds(off, 128)], vbuf.at[g], sl[g])

    def lin_wait(w, g):
        off = w * 128
        pltpu.make_async_copy(bcol.at[pl.ds(off, 128)], cbuf.at[g], sl[g]).wait()
        pltpu.make_async_copy(brow.at[pl.ds(off, 128)], rbuf.at[g], sl[g]).wait()
        pltpu.make_async_copy(bval.at[pl.ds(off, 128)], vbuf.at[g], sl[g]).wait()

    def wo_descr(a, b):
        return pltpu.make_async_copy(
            acc.at[pl.ds(a * _CB + sid * 256, 256)],
            out.at[pl.ds(b * _CB + sid * 256, 256)], wo[a])

    @pl.loop(0, 40, step=2)
    def _(i0):
        for a in range(2):
            i = i0 + a
            b = cid + 2 * i

            @pl.when(b < _NBK)
            def _():
                @pl.when(i >= 2)
                def _():
                    bp = b - 4

                    @pl.when(bp * _CB + sid * 256 < E)
                    def _():
                        wo_descr(a, bp).wait()
                for c in range(4):
                    pltpu.sync_copy(
                        zbuf, acc.at[pl.ds(a * _CB + sid * 256 + c * 64, 64)])
                plsc.subcore_barrier()

                start = _ext(offbuf, b)
                end = _ext(offbuf, b + 1)
                cnt = end - start
                s_t = start + ((sid * cnt) >> 4)
                e_t = start + (((sid + 1) * cnt) >> 4)
                w_lo = s_t >> 7
                w_hi = (e_t + 127) >> 7
                nw = w_hi - w_lo

                def sanitize(w, s):
                    for k in range(8):
                        gi = w * 128 + 16 * k + iota
                        r = rbuf.at[s][pl.ds(16 * k, 16)]
                        vv = vbuf.at[s][pl.ds(16 * k, 16)]
                        m = (gi >= s_t) & (gi < e_t)
                        lr = a * _CB + jnp.clip(r - b * _CB, 0, _CB - 1)
                        rbuf.at[s][pl.ds(16 * k, 16)] = lr
                        vbuf.at[s][pl.ds(16 * k, 16)] = jnp.where(m, vv, 0.0)

                @pl.when(nw >= 1)
                def _():
                    lin_start(w_lo, 0)

                @pl.when(nw >= 2)
                def _():
                    lin_start(w_lo + 1, 1)

                @pl.when(nw >= 3)
                def _():
                    lin_start(w_lo + 2, 2)

                @pl.when(nw >= 1)
                def _():
                    lin_wait(w_lo, 0)
                    sanitize(w_lo, 0)
                    pltpu.async_copy(xmat.at[cbuf.at[0]], gbuf.at[0], sg[0])

                @pl.loop(w_lo, w_hi, step=4)
                def _(q0):
                    for s in range(4):
                        w = q0 + s
                        g = s % 2
                        s1 = (s + 1) % 4
                        g1 = (s + 1) % 2

                        @pl.when(w < w_hi)
                        def _():
                            @pl.when(w + 1 < w_hi)
                            def _():
                                lin_wait(w + 1, s1)
                                sanitize(w + 1, s1)

                                @pl.when(w >= w_lo + 1)
                                def _():
                                    pltpu.make_async_copy(
                                        gbuf.at[g1], acc.at[rbuf.at[g1]],
                                        ss[g1]).wait()
                                pltpu.async_copy(
                                    xmat.at[cbuf.at[s1]], gbuf.at[g1], sg[g1])

                            pltpu.make_async_copy(
                                xmat.at[cbuf.at[s]], gbuf.at[g], sg[g]).wait()

                            @pl.loop(0, 128)
                            def _(j):
                                vs = plsc.load_gather(
                                    vbuf.at[s], [jnp.full((16,), j, jnp.int32)])
                                for k in range(8):
                                    gbuf.at[g][j, pl.ds(16 * k, 16)] = (
                                        gbuf.at[g][j, pl.ds(16 * k, 16)] * vs)
                            pltpu.async_copy(
                                gbuf.at[g], acc.at[rbuf.at[s]],
                                ss[g], add=True)

                            @pl.when(w + 3 < w_hi)
                            def _():
                                lin_start(w + 3, (s + 3) % 4)

                @pl.when(nw == 1)
                def _():
                    pltpu.make_async_copy(
                        gbuf.at[0], acc.at[rbuf.at[0]], ss[0]).wait()

                @pl.when(nw >= 2)
                def _():
                    pltpu.make_async_copy(
                        gbuf.at[0], acc.at[rbuf.at[0]], ss[0]).wait()
                    pltpu.make_async_copy(
                        gbuf.at[1], acc.at[rbuf.at[1]], ss[1]).wait()

                plsc.subcore_barrier()

                @pl.when(b * _CB + sid * 256 < E)
                def _():
                    wo_descr(a, b).start()

    b0 = cid + 76
    b1 = jnp.where(cid == 0, 78, 75)

    @pl.when(b0 * _CB + sid * 256 < E)
    def _():
        wo_descr(0, b0).wait()

    @pl.when(b1 * _CB + sid * 256 < E)
    def _():
        wo_descr(1, b1).wait()


_seg_call = pl.kernel(
    _seg_body,
    out_type=jax.ShapeDtypeStruct((E, D), jnp.float32),
    mesh=plsc.VectorSubcoreMesh(core_axis_name="c", subcore_axis_name="s",
                                num_cores=NC, num_subcores=NS),
    compiler_params=pltpu.CompilerParams(needs_layout_passes=False),
    scratch_types=[
        pltpu.VMEM_SHARED((2 * _CB, D), jnp.float32),
        pltpu.VMEM((128,), jnp.int32),
        pltpu.VMEM((4, 128), jnp.int32),
        pltpu.VMEM((4, 128), jnp.int32),
        pltpu.VMEM((4, 128), jnp.float32),
        pltpu.VMEM((2, 128, D), jnp.float32),
        pltpu.VMEM((64, D), jnp.float32),
        pltpu.SemaphoreType.DMA, pltpu.SemaphoreType.DMA,
        pltpu.SemaphoreType.DMA, pltpu.SemaphoreType.DMA,
        pltpu.SemaphoreType.DMA, pltpu.SemaphoreType.DMA,
        pltpu.SemaphoreType.DMA, pltpu.SemaphoreType.DMA,
        pltpu.SemaphoreType.DMA, pltpu.SemaphoreType.DMA,
    ],
)


def _pad_coo(rows, cols, vals, n_pad, row_mod):
    pad = n_pad - rows.shape[0]
    ar = jnp.arange(pad, dtype=jnp.int32)
    rows = jnp.concatenate([rows.astype(jnp.int32), ar % row_mod])
    cols = jnp.concatenate([cols.astype(jnp.int32), ar % E])
    vals = jnp.concatenate([vals, jnp.zeros((pad,), jnp.float32)])
    return rows, cols, vals


_BR = 1000


def _proj_body(x_ref, wi_ref, wu_ref, wd_ref, xi_ref, xu_ref, xd_ref):
    x = x_ref[...]
    xi_ref[...] = jnp.dot(x, wi_ref[...], preferred_element_type=jnp.float32)
    xu_ref[...] = jnp.dot(x, wu_ref[...], preferred_element_type=jnp.float32)
    xd_ref[...] = jnp.dot(x, wd_ref[...], preferred_element_type=jnp.float32)


def _project(x_1, W_id, W_up, W_down):
    grid = (E // _BR,)
    bs_x = pl.BlockSpec((_BR, D), lambda i: (i, 0))
    bs_w = pl.BlockSpec((D, D), lambda i: (0, 0))
    return pl.pallas_call(
        _proj_body,
        grid=grid,
        in_specs=[bs_x, bs_w, bs_w, bs_w],
        out_specs=[bs_x, bs_x, bs_x],
        out_shape=[jax.ShapeDtypeStruct((E, D), jnp.float32)] * 3,
    )(x_1, W_id, W_up, W_down)


def _relu_sum_body(a_ref, b_ref, c_ref, o_ref):
    o_ref[...] = jnp.maximum(a_ref[...] + b_ref[...] + c_ref[...], 0.0)


def _relu_sum(a, b, c):
    grid = (E // _BR,)
    bs = pl.BlockSpec((_BR, D), lambda i: (i, 0))
    return pl.pallas_call(
        _relu_sum_body,
        grid=grid,
        in_specs=[bs, bs, bs],
        out_specs=bs,
        out_shape=jax.ShapeDtypeStruct((E, D), jnp.float32),
    )(a, b, c)


def _add2_body(a_ref, b_ref, o_ref):
    o_ref[...] = a_ref[...] + b_ref[...]


def _add_partials(p):
    grid = (N // 1000,)
    bs = pl.BlockSpec((1000, D), lambda i: (i, 0))
    return pl.pallas_call(
        _add2_body,
        grid=grid,
        in_specs=[bs, bs],
        out_specs=bs,
        out_shape=jax.ShapeDtypeStruct((N, D), jnp.float32),
    )(p[:N], p[_NP:_NP + N])


def kernel(x_1, lap_up_indices, lap_up_values, lap_down_indices, lap_down_values,
           inc_rows, inc_cols, inc_values, y, W_id, W_up, W_down):
    xi, xu, xd = _project(x_1, W_id, W_up, W_down)

    ru, cu, vu = _pad_coo(lap_up_indices[0], lap_up_indices[1], lap_up_values,
                          _PAD_L, _CB)
    rd, cd, vd = _pad_coo(lap_down_indices[0], lap_down_indices[1],
                          lap_down_values, _PAD_L, _CB)
    bru, bcu, bvu, off_u, brd, bcd, bvd, off_d = _bin_call(ru, cu, vu, rd, cd, vd)
    h_up = _seg_call(bru, bcu, bvu, off_u, xu)
    h_down = _seg_call(brd, bcd, bvd, off_d, xd)
    h = _relu_sum(xi, h_up, h_down)

    ir, ic, iv = _pad_coo(inc_rows, inc_cols, inc_values, _PAD_I, N)
    partials = _inc_call(h, ir, ic, iv)
    x_0 = _add_partials(partials)
    return (x_0, y)


# speedup vs baseline: 1.9142x; 1.3349x over previous
import functools

import jax
import jax.numpy as jnp
from jax import lax
from jax.experimental import pallas as pl
from jax.experimental.pallas import tpu as pltpu
from jax.experimental.pallas import tpu_sc as plsc

E = 320000
N = 10000
D = 128

NC = 2
NS = 16
NW = NC * NS

_W = 128
_TI = 160
_PAD_I = NW * _TI * _W
_NP = 10240


def _inc_body(h_hbm, rows_hbm, cols_hbm, vals_hbm, out_hbm,
              acc, ibr, ibc, ibv, gbuf, zbuf,
              slin0, slin1, slin2, slin3, sg0, sg1, ss0, ss1):
    cid = lax.axis_index("c")
    sid = lax.axis_index("s")
    wid = sid * NC + cid
    base = wid * _TI * _W

    slin = (slin0, slin1, slin2, slin3)
    sg = (sg0, sg1)
    ss = (ss0, ss1)

    @pl.loop(0, 64)
    def _(i):
        for k in range(8):
            zbuf[i, pl.ds(16 * k, 16)] = jnp.zeros((16,), jnp.float32)
    for c in range(10):
        pltpu.sync_copy(zbuf, acc.at[pl.ds(sid * 640 + c * 64, 64)])
    plsc.subcore_barrier()

    def lin_start(w, s):
        off = base + w * _W
        pltpu.async_copy(rows_hbm.at[pl.ds(off, _W)], ibr.at[s], slin[s])
        pltpu.async_copy(cols_hbm.at[pl.ds(off, _W)], ibc.at[s], slin[s])
        pltpu.async_copy(vals_hbm.at[pl.ds(off, _W)], ibv.at[s], slin[s])

    def lin_wait(w, s):
        off = base + w * _W
        pltpu.make_async_copy(rows_hbm.at[pl.ds(off, _W)], ibr.at[s], slin[s]).wait()
        pltpu.make_async_copy(cols_hbm.at[pl.ds(off, _W)], ibc.at[s], slin[s]).wait()
        pltpu.make_async_copy(vals_hbm.at[pl.ds(off, _W)], ibv.at[s], slin[s]).wait()

    def gat_start(b, g):
        pltpu.async_copy(h_hbm.at[ibc.at[b]], gbuf.at[g], sg[g])

    def gat_wait(b, g):
        pltpu.make_async_copy(h_hbm.at[ibc.at[b]], gbuf.at[g], sg[g]).wait()

    def scat_start(b, g):
        pltpu.async_copy(gbuf.at[g], acc.at[ibr.at[b]], ss[g], add=True)

    def scat_wait(b, g):
        pltpu.make_async_copy(gbuf.at[g], acc.at[ibr.at[b]], ss[g]).wait()

    def mul(b, g):
        @pl.loop(0, _W)
        def _(j):
            vs = plsc.load_gather(ibv.at[b], [jnp.full((16,), j, jnp.int32)])
            for k in range(8):
                gbuf.at[g][j, pl.ds(16 * k, 16)] = (
                    gbuf.at[g][j, pl.ds(16 * k, 16)] * vs)

    lin_start(0, 0)
    lin_start(1, 1)
    lin_wait(0, 0)
    gat_start(0, 0)

    @pl.loop(0, _TI, step=4)
    def _(w0):
        for s in range(4):
            w = w0 + s
            b, g = s, s % 2
            gat_wait(b, g)

            @pl.when(w >= 1)
            def _():
                scat_wait((s + 3) % 4, (g + 1) % 2)

            @pl.when(w + 1 < _TI)
            def _():
                lin_wait(w + 1, (s + 1) % 4)
                gat_start((s + 1) % 4, (g + 1) % 2)

            mul(b, g)
            scat_start(b, g)

            @pl.when(w + 2 < _TI)
            def _():
                lin_start(w + 2, (s + 2) % 4)

    scat_wait((_TI - 1) % 4, (_TI - 1) % 2)
    plsc.subcore_barrier()

    for c in range(5):
        off = sid * 640 + c * 128
        pltpu.sync_copy(acc.at[pl.ds(off, 128)],
                        out_hbm.at[pl.ds(cid * _NP + off, 128)])


_inc_call = pl.kernel(
    _inc_body,
    out_type=jax.ShapeDtypeStruct((NC * _NP, D), jnp.float32),
    mesh=plsc.VectorSubcoreMesh(core_axis_name="c", subcore_axis_name="s",
                                num_cores=NC, num_subcores=NS),
    compiler_params=pltpu.CompilerParams(needs_layout_passes=False),
    scratch_types=[
        pltpu.VMEM_SHARED((_NP, D), jnp.float32),
        pltpu.VMEM((4, _W), jnp.int32),
        pltpu.VMEM((4, _W), jnp.int32),
        pltpu.VMEM((4, _W), jnp.float32),
        pltpu.VMEM((2, _W, D), jnp.float32),
        pltpu.VMEM((64, D), jnp.float32),
        pltpu.SemaphoreType.DMA, pltpu.SemaphoreType.DMA, pltpu.SemaphoreType.DMA,
        pltpu.SemaphoreType.DMA, pltpu.SemaphoreType.DMA, pltpu.SemaphoreType.DMA,
        pltpu.SemaphoreType.DMA, pltpu.SemaphoreType.DMA,
    ],
)


_WH = 1024
_NBK = 79
_NBP = 80
_CB = 4096
_TH = 98
_TP = 784
_PAD_L = 16 * _TH * _WH


def _bin_one(sid, rows, cols, vals, brow, bcol, off_out,
             ghist, lhist, hist, offs, hbuf, pbr, pbc, pbv, dbuf, sbuf,
             sh, sl, ss):
    base = sid * (_PAD_L // 16)
    iota = lax.iota(jnp.int32, 16)
    one = jnp.ones((16,), jnp.int32)
    zero16 = jnp.zeros((16,), jnp.int32)

    @pl.loop(0, _NBP)
    def _(b):
        hist[pl.ds(b * 16, 16)] = jnp.zeros((16,), jnp.int32)

    def h_start(w, s):
        pltpu.async_copy(rows.at[pl.ds(base + w * _WH, _WH)],
                         hbuf.at[pl.ds(s * _WH, _WH)], sh[s])

    def h_wait(w, s):
        pltpu.make_async_copy(rows.at[pl.ds(base + w * _WH, _WH)],
                              hbuf.at[pl.ds(s * _WH, _WH)], sh[s]).wait()

    h_start(0, 0)
    h_start(1, 1)

    @pl.loop(0, _TH, step=2)
    def _(w0):
        for s in range(2):
            w = w0 + s
            h_wait(w, s)

            @pl.loop(0, _WH // 16)
            def _(i):
                r = hbuf[pl.ds(s * _WH + i * 16, 16)]
                bkt = r >> 12
                plsc.addupdate_scatter(hist, [bkt * 16 + iota], one)

            @pl.when(w + 2 < _TH)
            def _():
                h_start(w + 2, s)

    pltpu.sync_copy(hist, ghist.at[pl.ds(sid * 1280, 1280)])
    plsc.subcore_barrier()
    pltpu.sync_copy(ghist, lhist)

    @pl.loop(0, _NBP, init_carry=jnp.zeros((16,), jnp.int32))
    def _(b, tv):
        @pl.when(sid == 0)
        def _():
            plsc.store_scatter(sbuf, [jnp.full((16,), b, jnp.int32)], tv,
                               mask=iota == 0)
        for t in range(16):
            v = lhist[pl.ds(t * 1280 + b * 16, 16)]
            cs = plsc.cumsum(v)

            @pl.when(sid == t)
            def _():
                offs[pl.ds(b * 16, 16)] = tv + (cs - v)

            tv = tv + lax.reduce_sum(jnp.where(iota == 15, cs, 0), (0,))
        return tv

    @pl.when(sid == 0)
    def _():
        pltpu.sync_copy(sbuf, off_out)
    plsc.subcore_barrier()

    def p_start(w, s):
        off = base + w * 128
        pltpu.async_copy(rows.at[pl.ds(off, 128)], pbr.at[s], sl[s])
        pltpu.async_copy(cols.at[pl.ds(off, 128)], pbc.at[s], sl[s])
        pltpu.async_copy(vals.at[pl.ds(off, 128)], pbv.at[s], sl[s])

    def p_wait(w, s):
        off = base + w * 128
        pltpu.make_async_copy(rows.at[pl.ds(off, 128)], pbr.at[s], sl[s]).wait()
        pltpu.make_async_copy(cols.at[pl.ds(off, 128)], pbc.at[s], sl[s]).wait()
        pltpu.make_async_copy(vals.at[pl.ds(off, 128)], pbv.at[s], sl[s]).wait()

    def sc_start(s):
        pltpu.async_copy(pbr.at[s], brow.at[dbuf.at[s]], ss[s])
        pltpu.async_copy(pbc.at[s], bcol.at[dbuf.at[s]], ss[s])

    def sc_wait(s):
        pltpu.make_async_copy(pbr.at[s], brow.at[dbuf.at[s]], ss[s]).wait()
        pltpu.make_async_copy(pbc.at[s], bcol.at[dbuf.at[s]], ss[s]).wait()

    p_start(0, 0)
    p_start(1, 1)

    @pl.loop(0, _TP, step=4)
    def _(w0):
        for s in range(4):
            w = w0 + s
            p_wait(w, s)
            for k in range(8):
                r = pbr.at[s][pl.ds(16 * k, 16)]
                bkt = r >> 12
                dest = plsc.load_gather(offs, [bkt * 16 + iota])
                plsc.store_scatter(offs, [bkt * 16 + iota], dest + one)
                dbuf.at[s][pl.ds(16 * k, 16)] = dest
                vbits = plsc.bitcast(pbv.at[s][pl.ds(16 * k, 16)], jnp.int32)
                pbr.at[s][pl.ds(16 * k, 16)] = (
                    (r & 0xFFF) | (vbits & jnp.int32(-65536)))
            sc_start(s)

            @pl.when(w >= 2)
            def _():
                sc_wait((s + 2) % 4)

            @pl.when(w + 2 < _TP)
            def _():
                p_start(w + 2, (s + 2) % 4)

    sc_wait((_TP - 2) % 4)
    sc_wait((_TP - 1) % 4)


def _bin_body(rows_u, cols_u, vals_u, rows_d, cols_d, vals_d,
              brow_u, bcol_u, off_u, brow_d, bcol_d, off_d,
              ghist, lhist, hist, offs, hbuf, pbr, pbc, pbv, dbuf, sbuf,
              sh0, sh1, sl0, sl1, sl2, sl3, ss0, ss1, ss2, ss3):
    cid = lax.axis_index("c")
    sid = lax.axis_index("s")
    sh = (sh0, sh1)
    sl = (sl0, sl1, sl2, sl3)
    ss = (ss0, ss1, ss2, ss3)

    @pl.when(cid == 0)
    def _():
        _bin_one(sid, rows_u, cols_u, vals_u, brow_u, bcol_u, off_u,
                 ghist, lhist, hist, offs, hbuf, pbr, pbc, pbv, dbuf, sbuf,
                 sh, sl, ss)

    @pl.when(cid == 1)
    def _():
        _bin_one(sid, rows_d, cols_d, vals_d, brow_d, bcol_d, off_d,
                 ghist, lhist, hist, offs, hbuf, pbr, pbc, pbv, dbuf, sbuf,
                 sh, sl, ss)


_bin_call = pl.kernel(
    _bin_body,
    out_type=[
        jax.ShapeDtypeStruct((_PAD_L,), jnp.int32),
        jax.ShapeDtypeStruct((_PAD_L,), jnp.int32),
        jax.ShapeDtypeStruct((128,), jnp.int32),
        jax.ShapeDtypeStruct((_PAD_L,), jnp.int32),
        jax.ShapeDtypeStruct((_PAD_L,), jnp.int32),
        jax.ShapeDtypeStruct((128,), jnp.int32),
    ],
    mesh=plsc.VectorSubcoreMesh(core_axis_name="c", subcore_axis_name="s",
                                num_cores=NC, num_subcores=NS),
    compiler_params=pltpu.CompilerParams(needs_layout_passes=False),
    scratch_types=[
        pltpu.VMEM_SHARED((NS * _NBP * 16,), jnp.int32),
        pltpu.VMEM((NS * _NBP * 16,), jnp.int32),
        pltpu.VMEM((_NBP * 16,), jnp.int32),
        pltpu.VMEM((_NBP * 16,), jnp.int32),
        pltpu.VMEM((2 * _WH,), jnp.int32),
        pltpu.VMEM((4, 128), jnp.int32),
        pltpu.VMEM((4, 128), jnp.int32),
        pltpu.VMEM((4, 128), jnp.float32),
        pltpu.VMEM((4, 128), jnp.int32),
        pltpu.VMEM((128,), jnp.int32),
        pltpu.SemaphoreType.DMA, pltpu.SemaphoreType.DMA,
        pltpu.SemaphoreType.DMA, pltpu.SemaphoreType.DMA,
        pltpu.SemaphoreType.DMA, pltpu.SemaphoreType.DMA,
        pltpu.SemaphoreType.DMA, pltpu.SemaphoreType.DMA,
        pltpu.SemaphoreType.DMA, pltpu.SemaphoreType.DMA,
    ],
)


def _ext(buf, j):
    iota = lax.iota(jnp.int32, 16)
    blk = (j >> 4) << 4
    v = buf[pl.ds(blk, 16)]
    return lax.reduce_sum(jnp.where(iota == (j & 15), v, 0), (0,))


def _seg_body(brow, bcol, offv, xmat, out,
              acc, offbuf, cbuf, rbuf, vbuf, gbuf, zbuf,
              sl0, sl1, sl2, sl3, sg0, sg1, ss0, ss1, wo0, wo1):
    cid = lax.axis_index("c")
    sid = lax.axis_index("s")
    iota = lax.iota(jnp.int32, 16)
    sl = (sl0, sl1, sl2, sl3)
    sg = (sg0, sg1)
    ss = (ss0, ss1)
    wo = (wo0, wo1)

    pltpu.sync_copy(offv, offbuf)

    @pl.loop(0, 64)
    def _(i):
        for k in range(8):
            zbuf[i, pl.ds(16 * k, 16)] = jnp.zeros((16,), jnp.float32)

    def lin_start(w, g):
        off = w * 128
        pltpu.async_copy(bcol.at[pl.ds(off, 128)], cbuf.at[g], sl[g])
        pltpu.async_copy(brow.at[pl.ds(off, 128)], rbuf.at[g], sl[g])

    def lin_wait(w, g):
        off = w * 128
        pltpu.make_async_copy(bcol.at[pl.ds(off, 128)], cbuf.at[g], sl[g]).wait()
        pltpu.make_async_copy(brow.at[pl.ds(off, 128)], rbuf.at[g], sl[g]).wait()

    def wo_descr(a, b):
        return pltpu.make_async_copy(
            acc.at[pl.ds(a * _CB + sid * 256, 256)],
            out.at[pl.ds(b * _CB + sid * 256, 256)], wo[a])

    @pl.loop(0, 40, step=2)
    def _(i0):
        for a in range(2):
            i = i0 + a
            b = cid + 2 * i

            @pl.when(b < _NBK)
            def _():
                @pl.when(i >= 2)
                def _():
                    bp = b - 4

                    @pl.when(bp * _CB + sid * 256 < E)
                    def _():
                        wo_descr(a, bp).wait()
                for c in range(4):
                    pltpu.sync_copy(
                        zbuf, acc.at[pl.ds(a * _CB + sid * 256 + c * 64, 64)])
                plsc.subcore_barrier()

                start = _ext(offbuf, b)
                end = _ext(offbuf, b + 1)
                cnt = end - start
                s_t = start + ((sid * cnt) >> 4)
                e_t = start + (((sid + 1) * cnt) >> 4)
                w_lo = s_t >> 7
                w_hi = (e_t + 127) >> 7
                nw = w_hi - w_lo

                def sanitize(w, s):
                    for k in range(8):
                        gi = w * 128 + 16 * k + iota
                        p = rbuf.at[s][pl.ds(16 * k, 16)]
                        vv = plsc.bitcast(p & jnp.int32(-65536), jnp.float32)
                        m = (gi >= s_t) & (gi < e_t)
                        rbuf.at[s][pl.ds(16 * k, 16)] = a * _CB + (p & 0xFFF)
                        vbuf.at[s][pl.ds(16 * k, 16)] = jnp.where(m, vv, 0.0)

                @pl.when(nw >= 1)
                def _():
                    lin_start(w_lo, 0)

                @pl.when(nw >= 2)
                def _():
                    lin_start(w_lo + 1, 1)

                @pl.when(nw >= 3)
                def _():
                    lin_start(w_lo + 2, 2)

                @pl.when(nw >= 1)
                def _():
                    lin_wait(w_lo, 0)
                    sanitize(w_lo, 0)
                    pltpu.async_copy(xmat.at[cbuf.at[0]], gbuf.at[0], sg[0])

                @pl.loop(w_lo, w_hi, step=4)
                def _(q0):
                    for s in range(4):
                        w = q0 + s
                        g = s % 2
                        s1 = (s + 1) % 4
                        g1 = (s + 1) % 2

                        @pl.when(w < w_hi)
                        def _():
                            @pl.when(w + 1 < w_hi)
                            def _():
                                lin_wait(w + 1, s1)
                                sanitize(w + 1, s1)

                                @pl.when(w >= w_lo + 1)
                                def _():
                                    pltpu.make_async_copy(
                                        gbuf.at[g1], acc.at[rbuf.at[g1]],
                                        ss[g1]).wait()
                                pltpu.async_copy(
                                    xmat.at[cbuf.at[s1]], gbuf.at[g1], sg[g1])

                            pltpu.make_async_copy(
                                xmat.at[cbuf.at[s]], gbuf.at[g], sg[g]).wait()

                            @pl.loop(0, 128)
                            def _(j):
                                vs = plsc.load_gather(
                                    vbuf.at[s], [jnp.full((16,), j, jnp.int32)])
                                for k in range(8):
                                    gbuf.at[g][j, pl.ds(16 * k, 16)] = (
                                        gbuf.at[g][j, pl.ds(16 * k, 16)] * vs)
                            pltpu.async_copy(
                                gbuf.at[g], acc.at[rbuf.at[s]],
                                ss[g], add=True)

                            @pl.when(w + 3 < w_hi)
                            def _():
                                lin_start(w + 3, (s + 3) % 4)

                @pl.when(nw == 1)
                def _():
                    pltpu.make_async_copy(
                        gbuf.at[0], acc.at[rbuf.at[0]], ss[0]).wait()

                @pl.when(nw >= 2)
                def _():
                    pltpu.make_async_copy(
                        gbuf.at[0], acc.at[rbuf.at[0]], ss[0]).wait()
                    pltpu.make_async_copy(
                        gbuf.at[1], acc.at[rbuf.at[1]], ss[1]).wait()

                plsc.subcore_barrier()

                @pl.when(b * _CB + sid * 256 < E)
                def _():
                    wo_descr(a, b).start()

    b0 = cid + 76
    b1 = jnp.where(cid == 0, 78, 75)

    @pl.when(b0 * _CB + sid * 256 < E)
    def _():
        wo_descr(0, b0).wait()

    @pl.when(b1 * _CB + sid * 256 < E)
    def _():
        wo_descr(1, b1).wait()


_seg_call = pl.kernel(
    _seg_body,
    out_type=jax.ShapeDtypeStruct((E, D), jnp.float32),
    mesh=plsc.VectorSubcoreMesh(core_axis_name="c", subcore_axis_name="s",
                                num_cores=NC, num_subcores=NS),
    compiler_params=pltpu.CompilerParams(needs_layout_passes=False),
    scratch_types=[
        pltpu.VMEM_SHARED((2 * _CB, D), jnp.float32),
        pltpu.VMEM((128,), jnp.int32),
        pltpu.VMEM((4, 128), jnp.int32),
        pltpu.VMEM((4, 128), jnp.int32),
        pltpu.VMEM((4, 128), jnp.float32),
        pltpu.VMEM((2, 128, D), jnp.float32),
        pltpu.VMEM((64, D), jnp.float32),
        pltpu.SemaphoreType.DMA, pltpu.SemaphoreType.DMA,
        pltpu.SemaphoreType.DMA, pltpu.SemaphoreType.DMA,
        pltpu.SemaphoreType.DMA, pltpu.SemaphoreType.DMA,
        pltpu.SemaphoreType.DMA, pltpu.SemaphoreType.DMA,
        pltpu.SemaphoreType.DMA, pltpu.SemaphoreType.DMA,
    ],
)


def _pad_coo(rows, cols, vals, n_pad, row_mod):
    pad = n_pad - rows.shape[0]
    ar = jnp.arange(pad, dtype=jnp.int32)
    rows = jnp.concatenate([rows.astype(jnp.int32), ar % row_mod])
    cols = jnp.concatenate([cols.astype(jnp.int32), ar % E])
    vals = jnp.concatenate([vals, jnp.zeros((pad,), jnp.float32)])
    return rows, cols, vals


_BR = 1000


def _proj_body(x_ref, wi_ref, wu_ref, wd_ref, xi_ref, xu_ref, xd_ref):
    x = x_ref[...]
    xi_ref[...] = jnp.dot(x, wi_ref[...], preferred_element_type=jnp.float32)
    xu_ref[...] = jnp.dot(x, wu_ref[...], preferred_element_type=jnp.float32)
    xd_ref[...] = jnp.dot(x, wd_ref[...], preferred_element_type=jnp.float32)


def _project(x_1, W_id, W_up, W_down):
    grid = (E // _BR,)
    bs_x = pl.BlockSpec((_BR, D), lambda i: (i, 0))
    bs_w = pl.BlockSpec((D, D), lambda i: (0, 0))
    return pl.pallas_call(
        _proj_body,
        grid=grid,
        in_specs=[bs_x, bs_w, bs_w, bs_w],
        out_specs=[bs_x, bs_x, bs_x],
        out_shape=[jax.ShapeDtypeStruct((E, D), jnp.float32)] * 3,
    )(x_1, W_id, W_up, W_down)


def _relu_sum_body(a_ref, b_ref, c_ref, o_ref):
    o_ref[...] = jnp.maximum(a_ref[...] + b_ref[...] + c_ref[...], 0.0)


def _relu_sum(a, b, c):
    grid = (E // _BR,)
    bs = pl.BlockSpec((_BR, D), lambda i: (i, 0))
    return pl.pallas_call(
        _relu_sum_body,
        grid=grid,
        in_specs=[bs, bs, bs],
        out_specs=bs,
        out_shape=jax.ShapeDtypeStruct((E, D), jnp.float32),
    )(a, b, c)


def _add2_body(a_ref, b_ref, o_ref):
    o_ref[...] = a_ref[...] + b_ref[...]


def _add_partials(p):
    grid = (N // 1000,)
    bs = pl.BlockSpec((1000, D), lambda i: (i, 0))
    return pl.pallas_call(
        _add2_body,
        grid=grid,
        in_specs=[bs, bs],
        out_specs=bs,
        out_shape=jax.ShapeDtypeStruct((N, D), jnp.float32),
    )(p[:N], p[_NP:_NP + N])


def kernel(x_1, lap_up_indices, lap_up_values, lap_down_indices, lap_down_values,
           inc_rows, inc_cols, inc_values, y, W_id, W_up, W_down):
    xi, xu, xd = _project(x_1, W_id, W_up, W_down)

    ru, cu, vu = _pad_coo(lap_up_indices[0], lap_up_indices[1], lap_up_values,
                          _PAD_L, _CB)
    rd, cd, vd = _pad_coo(lap_down_indices[0], lap_down_indices[1],
                          lap_down_values, _PAD_L, _CB)
    bru, bcu, off_u, brd, bcd, off_d = _bin_call(ru, cu, vu, rd, cd, vd)
    h_up = _seg_call(bru, bcu, off_u, xu)
    h_down = _seg_call(brd, bcd, off_d, xd)
    h = _relu_sum(xi, h_up, h_down)

    ir, ic, iv = _pad_coo(inc_rows, inc_cols, inc_values, _PAD_I, N)
    partials = _inc_call(h, ir, ic, iv)
    x_0 = _add_partials(partials)
    return (x_0, y)
